# Initial kernel scaffold; baseline (speedup 1.0000x reference)
import jax
import jax.numpy as jnp
from jax.experimental import pallas as pl


def kernel(x, edge_index, W_proj, W1_l, b1_l, W1_r, W2_l, b2_l, W2_r):
    raise NotImplementedError("write your pallas kernel here")


# speedup vs baseline: 4.3158x; 4.3158x over previous
import functools

import jax
import jax.numpy as jnp
from jax import lax
from jax.experimental import pallas as pl
from jax.experimental.pallas import tpu as pltpu
from jax.experimental.pallas import tpu_sc as plsc

N = 10000
E = 160000
D = 256
H = 128
NC = 2
NS = 16
R = 10240
CH = 128
NCHUNK = 79
EPT = NCHUNK * CH
EPAD = EPT * NS
RPT = R // NS
BLK = 1024
GRID = R // BLK


def _proj_body(x_ref, wp_ref, h_ref, h3_ref):
    x = x_ref[...]
    h = jnp.dot(x, wp_ref[...].T, preferred_element_type=jnp.float32)
    h_ref[...] = h
    h3_ref[0] = h[:, 0:H]
    h3_ref[1] = h[:, H:D]


_proj_call = pl.pallas_call(
    _proj_body,
    grid=(GRID,),
    in_specs=[
        pl.BlockSpec((BLK, D), lambda i: (i, 0)),
        pl.BlockSpec((D, D), lambda i: (0, 0)),
    ],
    out_specs=[
        pl.BlockSpec((BLK, D), lambda i: (i, 0)),
        pl.BlockSpec((NC, BLK, H), lambda i: (0, i, 0)),
    ],
    out_shape=[
        jax.ShapeDtypeStruct((R, D), jnp.float32),
        jax.ShapeDtypeStruct((NC, R, H), jnp.float32),
    ],
)


def _sage_mid_body(agg_ref, cnt_ref, h_ref, wl_ref, bl_ref, wr_ref,
                   o_ref, o3_ref):
    s = jnp.concatenate([agg_ref[0], agg_ref[1]], axis=1)
    cnt = cnt_ref[...]
    mean = s / jnp.maximum(cnt, 1.0)
    o = jnp.dot(mean, wl_ref[...].T, preferred_element_type=jnp.float32)
    o = o + bl_ref[...]
    o = o + jnp.dot(h_ref[...], wr_ref[...].T, preferred_element_type=jnp.float32)
    o = jnp.maximum(o, 0.0)
    o_ref[...] = o
    o3_ref[0] = o[:, 0:H]
    o3_ref[1] = o[:, H:D]


_sage_mid_call = pl.pallas_call(
    _sage_mid_body,
    grid=(GRID,),
    in_specs=[
        pl.BlockSpec((NC, BLK, H), lambda i: (0, i, 0)),
        pl.BlockSpec((BLK, 1), lambda i: (i, 0)),
        pl.BlockSpec((BLK, D), lambda i: (i, 0)),
        pl.BlockSpec((D, D), lambda i: (0, 0)),
        pl.BlockSpec((1, D), lambda i: (0, 0)),
        pl.BlockSpec((D, D), lambda i: (0, 0)),
    ],
    out_specs=[
        pl.BlockSpec((BLK, D), lambda i: (i, 0)),
        pl.BlockSpec((NC, BLK, H), lambda i: (0, i, 0)),
    ],
    out_shape=[
        jax.ShapeDtypeStruct((R, D), jnp.float32),
        jax.ShapeDtypeStruct((NC, R, H), jnp.float32),
    ],
)


def _sage_fin_body(agg_ref, cnt_ref, h_ref, wl_ref, bl_ref, wr_ref, o_ref):
    s = jnp.concatenate([agg_ref[0], agg_ref[1]], axis=1)
    cnt = cnt_ref[...]
    mean = s / jnp.maximum(cnt, 1.0)
    o = jnp.dot(mean, wl_ref[...].T, preferred_element_type=jnp.float32)
    o = o + bl_ref[...]
    o = o + jnp.dot(h_ref[...], wr_ref[...].T, preferred_element_type=jnp.float32)
    norm = jnp.sqrt(jnp.sum(o * o, axis=1, keepdims=True))
    o_ref[...] = o / jnp.maximum(norm, 1e-12)


_sage_fin_call = pl.pallas_call(
    _sage_fin_body,
    grid=(GRID,),
    in_specs=[
        pl.BlockSpec((NC, BLK, H), lambda i: (0, i, 0)),
        pl.BlockSpec((BLK, 1), lambda i: (i, 0)),
        pl.BlockSpec((BLK, D), lambda i: (i, 0)),
        pl.BlockSpec((D, D), lambda i: (0, 0)),
        pl.BlockSpec((1, D), lambda i: (0, 0)),
        pl.BlockSpec((D, D), lambda i: (0, 0)),
    ],
    out_specs=pl.BlockSpec((BLK, D), lambda i: (i, 0)),
    out_shape=jax.ShapeDtypeStruct((R, D), jnp.float32),
)


def _sc_agg_body(h3_hbm, src_hbm, dst_hbm, zrow_hbm, ones_hbm,
                 outf_hbm, outc_hbm,
                 src_v, dst_v, rows_v, ones_v, cnt_v, accf, accc, sem):
    c = lax.axis_index("c")
    s = lax.axis_index("s")

    pltpu.sync_copy(src_hbm.at[s], src_v)
    pltpu.sync_copy(dst_hbm.at[s], dst_v)
    pltpu.sync_copy(ones_hbm, ones_v)

    off = c * R

    def _adj(i, _):
        j = i // (CH // 16)
        k = i % (CH // 16)
        v = src_v[j, pl.ds(k * 16, 16)]
        src_v[j, pl.ds(k * 16, 16)] = v + off
        return 0

    lax.fori_loop(0, NCHUNK * (CH // 16), _adj, 0, unroll=8)

    pltpu.sync_copy(zrow_hbm, rows_v)

    def _zero(i, _):
        pltpu.sync_copy(rows_v, accf.at[pl.ds(s * RPT + i * CH, CH)])
        pltpu.sync_copy(rows_v.at[0], accc.at[pl.ds(s * RPT + i * CH, CH)])
        return 0

    lax.fori_loop(0, RPT // CH, _zero, 0)
    plsc.subcore_barrier()

    def _body(j, _):
        pltpu.async_copy(h3_hbm.at[src_v.at[j]], rows_v, sem).wait()
        pltpu.sync_copy(rows_v, accf.at[dst_v.at[j]], add=True)
        pltpu.sync_copy(ones_v, accc.at[dst_v.at[j]], add=True)
        return 0

    lax.fori_loop(0, NCHUNK, _body, 0)
    plsc.subcore_barrier()

    def _wb(i, _):
        base = s * RPT + i * CH
        pltpu.sync_copy(accf.at[pl.ds(base, CH)], rows_v)
        pltpu.sync_copy(rows_v, outf_hbm.at[pl.ds(off + base, CH)])
        return 0

    lax.fori_loop(0, RPT // CH, _wb, 0)
    pltpu.sync_copy(accc.at[pl.ds(s * RPT, RPT)], cnt_v)
    pltpu.sync_copy(cnt_v, outc_hbm.at[c].at[pl.ds(s * RPT, RPT)])


@functools.cache
def _sc_agg_call():
    return pl.kernel(
        _sc_agg_body,
        out_type=[
            jax.ShapeDtypeStruct((NC * R, H), jnp.float32),
            jax.ShapeDtypeStruct((NC, R), jnp.float32),
        ],
        mesh=plsc.VectorSubcoreMesh(core_axis_name="c", subcore_axis_name="s",
                                    num_cores=NC, num_subcores=NS),
        scratch_types=[
            pltpu.VMEM((NCHUNK, CH), jnp.int32),
            pltpu.VMEM((NCHUNK, CH), jnp.int32),
            pltpu.VMEM((CH, H), jnp.float32),
            pltpu.VMEM((CH,), jnp.float32),
            pltpu.VMEM((RPT,), jnp.float32),
            pltpu.VMEM_SHARED((R, H), jnp.float32),
            pltpu.VMEM_SHARED((R,), jnp.float32),
            pltpu.SemaphoreType.DMA,
        ],
    )


def kernel(x, edge_index, W_proj, W1_l, b1_l, W1_r, W2_l, b2_l, W2_r):
    x = x.astype(jnp.float32)
    src = edge_index[0].astype(jnp.int32)
    dst = edge_index[1].astype(jnp.int32)

    pad = EPAD - E
    src_p = jnp.concatenate([src, jnp.zeros((pad,), jnp.int32)])
    dst_p = jnp.concatenate([dst, jnp.full((pad,), N, jnp.int32)])
    src3 = src_p.reshape(NS, NCHUNK, CH)
    dst3 = dst_p.reshape(NS, NCHUNK, CH)
    zrow = jnp.zeros((CH, H), jnp.float32)
    onesv = jnp.ones((CH,), jnp.float32)

    x_p = jnp.pad(x, ((0, R - N), (0, 0)))
    b1 = b1_l.reshape(1, D).astype(jnp.float32)
    b2 = b2_l.reshape(1, D).astype(jnp.float32)

    h, h3 = _proj_call(x_p, W_proj.astype(jnp.float32))
    agg1, cnt1 = _sc_agg_call()(h3.reshape(NC * R, H), src3, dst3, zrow, onesv)
    cnt = cnt1[0].reshape(R, 1)
    h1, h13 = _sage_mid_call(agg1.reshape(NC, R, H), cnt, h,
                             W1_l.astype(jnp.float32), b1,
                             W1_r.astype(jnp.float32))
    agg2, _ = _sc_agg_call()(h13.reshape(NC * R, H), src3, dst3, zrow, onesv)
    out = _sage_fin_call(agg2.reshape(NC, R, H), cnt, h1,
                         W2_l.astype(jnp.float32), b2,
                         W2_r.astype(jnp.float32))
    return out[:N]
